# baseline (speedup 1.0000x reference)
import functools

import jax
import jax.numpy as jnp
from jax import lax
from jax.experimental import pallas as pl
from jax.experimental.pallas import tpu as pltpu
from jax.experimental.pallas import tpu_sc as plsc

E = 16
G = 4
EPG = E // G
K = 2
BLK = 1024

NC = 2
NS = 16
NW = NC * NS
L = 16


def _scores_kernel(w_ref, x_ref, s_ref):
    s_ref[...] = jax.lax.dot_general(
        w_ref[...], x_ref[...], (((1,), (1,)), ((), ())),
        preferred_element_type=jnp.float32)


def _route_kernel(tpw, s_hbm, wout_hbm, iout_hbm, s_v, wo_v, io_v):
    wid = lax.axis_index("s") * NC + lax.axis_index("c")
    base = wid * tpw
    pltpu.sync_copy(s_hbm.at[:, pl.ds(base, tpw)], s_v)

    neg = jnp.full((L,), -jnp.inf, dtype=jnp.float32)

    def maxtree(xs):
        xs = list(xs)
        while len(xs) > 1:
            xs = [jnp.maximum(xs[i], xs[i + 1])
                  for i in range(0, len(xs) - 1, 2)] + (
                      [xs[-1]] if len(xs) % 2 else [])
        return xs[0]

    def sumtree(xs):
        xs = list(xs)
        while len(xs) > 1:
            xs = [xs[i] + xs[i + 1]
                  for i in range(0, len(xs) - 1, 2)] + (
                      [xs[-1]] if len(xs) % 2 else [])
        return xs[0]

    def argmaxtree(vals, idxs):
        vs, ids = list(vals), list(idxs)
        while len(vs) > 1:
            nv, ni = [], []
            for i in range(0, len(vs) - 1, 2):
                take = vs[i] >= vs[i + 1]
                nv.append(jnp.where(take, vs[i], vs[i + 1]))
                ni.append(jnp.where(take, ids[i], ids[i + 1]))
            if len(vs) % 2:
                nv.append(vs[-1])
                ni.append(ids[-1])
            vs, ids = nv, ni
        return vs[0], ids[0]

    UNROLL = 2

    def body(t, carry):
        for u in range(UNROLL):
            _group(pl.multiple_of(t * (UNROLL * L) + u * L, L))
        return carry

    def _group(off):
        vs = [s_v[e, pl.ds(off, L)] for e in range(E)]
        m = maxtree(vs)
        ex = [jnp.exp(v - m) for v in vs]
        s = sumtree(ex)
        p = [v / s for v in ex]
        gs = [maxtree(p[g * EPG:(g + 1) * EPG]) for g in range(G)]
        gconst = [jnp.full((L,), g, dtype=jnp.int32) for g in range(G)]
        _, g1 = argmaxtree(gs, gconst)
        gs2 = [jnp.where(g1 == gconst[g], neg, gs[g]) for g in range(G)]
        _, g2 = argmaxtree(gs2, gconst)
        sel = []
        for e in range(E):
            ge = gconst[e // EPG]
            allowed = (g1 == ge) | (g2 == ge)
            sel.append(jnp.where(allowed, p[e], neg))
        econst = [jnp.full((L,), e, dtype=jnp.int32) for e in range(E)]
        m1, i1 = argmaxtree(sel, econst)
        sel2 = [jnp.where(i1 == econst[e], neg, sel[e]) for e in range(E)]
        m2, i2 = argmaxtree(sel2, econst)
        wo_v[0, pl.ds(off, L)] = m1
        wo_v[1, pl.ds(off, L)] = m2
        io_v[0, pl.ds(off, L)] = i1
        io_v[1, pl.ds(off, L)] = i2

    lax.fori_loop(0, tpw // (UNROLL * L), body, 0)
    pltpu.sync_copy(wo_v, wout_hbm.at[:, pl.ds(base, tpw)])
    pltpu.sync_copy(io_v, iout_hbm.at[:, pl.ds(base, tpw)])


@jax.jit
def kernel(x, W):
    n, d = x.shape
    scores_t = pl.pallas_call(
        _scores_kernel,
        grid=(n // BLK,),
        in_specs=[pl.BlockSpec((E, d), lambda i: (0, 0)),
                  pl.BlockSpec((BLK, d), lambda i: (i, 0))],
        out_specs=pl.BlockSpec((E, BLK), lambda i: (0, i)),
        out_shape=jax.ShapeDtypeStruct((E, n), jnp.float32),
    )(W, x)

    tpw = n // NW
    route = pl.kernel(
        functools.partial(_route_kernel, tpw),
        out_type=[jax.ShapeDtypeStruct((K, n), jnp.float32),
                  jax.ShapeDtypeStruct((K, n), jnp.int32)],
        mesh=plsc.VectorSubcoreMesh(core_axis_name="c", subcore_axis_name="s"),
        scratch_types=[pltpu.VMEM((E, tpw), jnp.float32),
                       pltpu.VMEM((K, tpw), jnp.float32),
                       pltpu.VMEM((K, tpw), jnp.int32)],
    )
    wout, iout = route(scores_t)
    return wout.T, iout.T
